# baseline (device time: 20316 ns/iter reference)
import functools

import jax
import jax.numpy as jnp
from jax import lax
from jax.experimental import pallas as pl
from jax.experimental.pallas import tpu as pltpu

N_DEV = 4
BLK = 64


def kernel(x, Wq, K_ext, V_ext, Wo):
    B, S_loc, D = x.shape
    _, _, H, Dh = K_ext.shape
    HD = H * Dh
    Dout = Wo.shape[1]

    def body(x_ref, wq_ref, k_ref, v_ref, wo_ref, out_ref,
             ks_ref, vs_ref, kg_ref, vg_ref,
             k_send, k_recv, v_send, v_recv):
        my = lax.axis_index("i")
        left = (my + N_DEV - 1) % N_DEV
        right = (my + 1) % N_DEV
        diag = (my + 2) % N_DEV
        peers = (left, right, diag)

        barrier = pltpu.get_barrier_semaphore()
        for nbr in peers:
            pl.semaphore_signal(barrier, inc=1, device_id=(nbr,),
                                device_id_type=pl.DeviceIdType.MESH)

        sends = []

        def send(src_ref, dst_ref, send_sems, sidx, recv_sems, ridx, tgt):
            r = pltpu.make_async_remote_copy(
                src_ref=src_ref, dst_ref=dst_ref,
                send_sem=send_sems.at[sidx], recv_sem=recv_sems.at[ridx],
                device_id=(tgt,), device_id_type=pl.DeviceIdType.MESH)
            r.start()
            sends.append(r)

        ks_ref[...] = k_ref[...].astype(jnp.float8_e4m3fn)
        vs_ref[...] = v_ref[...].astype(jnp.bfloat16)
        pl.semaphore_wait(barrier, len(peers))
        for idx, tgt in enumerate((left, right)):
            send(ks_ref, kg_ref.at[my], k_send, idx, k_recv, my * B, tgt)
            send(vs_ref, vg_ref.at[my], v_send, idx, v_recv, my * B, tgt)

        x2 = x_ref[...].reshape(B * S_loc, D)
        q2 = (jnp.dot(x2, wq_ref[...],
                      preferred_element_type=jnp.float32)
              * (0.125 * 1.4426950408889634)).astype(jnp.bfloat16)
        q_all = [q2[b * S_loc:(b + 1) * S_loc] for b in range(B)]

        row = lax.broadcasted_iota(jnp.int32, (S_loc, S_loc), 0)
        col = lax.broadcasted_iota(jnp.int32, (S_loc, S_loc), 1)
        qb = (my * S_loc + row) // BLK

        acc = [[jnp.zeros((S_loc, Dh), jnp.float32)
                for _ in range(H)] for _ in range(B)]
        lsum = [[jnp.zeros((S_loc, 1), jnp.float32)
                 for _ in range(H)] for _ in range(B)]

        def block_mask(origin):
            kb = (origin * S_loc + col) // BLK
            return (qb == kb) | (kb == 0) | ((qb + kb) % 3 == 0)

        def process_half(m, b, kc2, vc2):
            for hh in range(H):
                sl = slice(hh * Dh, (hh + 1) * Dh)
                s = jnp.dot(q_all[b][:, sl], kc2[:, sl].T,
                            preferred_element_type=jnp.float32)
                e = jnp.where(m, jnp.exp2(s), 0.0)
                lsum[b][hh] += jnp.sum(e, axis=-1, keepdims=True)
                acc[b][hh] += jnp.dot(e.astype(jnp.bfloat16), vc2[:, sl],
                                      preferred_element_type=jnp.float32)

        m_own = block_mask(my)
        for b in range(B):
            process_half(m_own, b, k_ref[b].astype(jnp.bfloat16),
                         v_ref[b].astype(jnp.bfloat16))

        def wait_recv(origin, half_slice, ridx):
            rk = pltpu.make_async_remote_copy(
                src_ref=kg_ref.at[origin, half_slice],
                dst_ref=kg_ref.at[origin, half_slice],
                send_sem=k_send.at[0], recv_sem=k_recv.at[ridx],
                device_id=(my,), device_id_type=pl.DeviceIdType.MESH)
            rv = pltpu.make_async_remote_copy(
                src_ref=vg_ref.at[origin, half_slice],
                dst_ref=vg_ref.at[origin, half_slice],
                send_sem=v_send.at[0], recv_sem=v_recv.at[ridx],
                device_id=(my,), device_id_type=pl.DeviceIdType.MESH)
            rk.wait_recv()
            rv.wait_recv()

        wait_recv(left, slice(None), left * B)
        send(kg_ref.at[left, 0], kg_ref.at[left, 0],
             k_send, 2, k_recv, left * B, right)
        send(vg_ref.at[left, 0], vg_ref.at[left, 0],
             v_send, 2, v_recv, left * B, right)
        m_l = block_mask(left)
        for b in range(B):
            process_half(m_l, b, kg_ref[left, b].astype(jnp.bfloat16),
                         vg_ref[left, b])

        wait_recv(right, slice(None), right * B)
        send(kg_ref.at[right, 1], kg_ref.at[right, 1],
             k_send, 3, k_recv, right * B + 1, left)
        send(vg_ref.at[right, 1], vg_ref.at[right, 1],
             v_send, 3, v_recv, right * B + 1, left)
        m_r = block_mask(right)
        for b in range(B):
            process_half(m_r, b, kg_ref[right, b].astype(jnp.bfloat16),
                         vg_ref[right, b])

        m_d = block_mask(diag)
        wait_recv(diag, 0, diag * B)
        process_half(m_d, 0, kg_ref[diag, 0].astype(jnp.bfloat16),
                     vg_ref[diag, 0])
        wait_recv(diag, 1, diag * B + 1)
        process_half(m_d, 1, kg_ref[diag, 1].astype(jnp.bfloat16),
                     vg_ref[diag, 1])

        @functools.partial(pl.run_scoped,
                           exit_sem=pltpu.SemaphoreType.REGULAR)
        def _(exit_sem):
            for nbr in peers:
                pl.semaphore_signal(exit_sem, inc=1, device_id=(nbr,),
                                    device_id_type=pl.DeviceIdType.MESH)

            for b in range(B):
                ctx = jnp.concatenate(
                    [acc[b][hh] / lsum[b][hh] for hh in range(H)], axis=-1)
                out_ref[b] = jnp.dot(ctx, wo_ref[...],
                                     preferred_element_type=jnp.float32)

            for r in sends:
                r.wait_send()
            pl.semaphore_wait(exit_sem, len(peers))

    return pl.pallas_call(
        body,
        out_shape=jax.ShapeDtypeStruct((B, S_loc, Dout), jnp.float32),
        in_specs=[pl.BlockSpec(memory_space=pltpu.VMEM)] * 5,
        out_specs=pl.BlockSpec(memory_space=pltpu.VMEM),
        scratch_shapes=[
            pltpu.VMEM((B, S_loc, HD), jnp.float8_e4m3fn),
            pltpu.VMEM((B, S_loc, HD), jnp.bfloat16),
            pltpu.VMEM((N_DEV, B, S_loc, HD), jnp.float8_e4m3fn),
            pltpu.VMEM((N_DEV, B, S_loc, HD), jnp.bfloat16),
            pltpu.SemaphoreType.DMA((3 * B,)),
            pltpu.SemaphoreType.DMA((N_DEV * B,)),
            pltpu.SemaphoreType.DMA((3 * B,)),
            pltpu.SemaphoreType.DMA((N_DEV * B,)),
        ],
        compiler_params=pltpu.CompilerParams(collective_id=0),
    )(x, Wq, K_ext.reshape(B, S_loc, HD), V_ext.reshape(B, S_loc, HD), Wo)


# device time: 20121 ns/iter; 1.0097x vs baseline; 1.0097x over previous
import functools

import jax
import jax.numpy as jnp
from jax import lax
from jax.experimental import pallas as pl
from jax.experimental.pallas import tpu as pltpu

N_DEV = 4
BLK = 64


def kernel(x, Wq, K_ext, V_ext, Wo):
    B, S_loc, D = x.shape
    _, _, H, Dh = K_ext.shape
    HD = H * Dh
    Dout = Wo.shape[1]

    def body(x_ref, wq_ref, k_ref, v_ref, wo_ref, out_ref,
             xv_ref, wqv_ref, wov_ref, ks_ref, vs_ref, kg_ref, vg_ref,
             load_sems, k_send, k_recv, v_send, v_recv):
        my = lax.axis_index("i")
        left = (my + N_DEV - 1) % N_DEV
        right = (my + 1) % N_DEV
        diag = (my + 2) % N_DEV
        peers = (left, right, diag)

        cx = pltpu.make_async_copy(x_ref, xv_ref, load_sems.at[0])
        cq = pltpu.make_async_copy(wq_ref, wqv_ref, load_sems.at[1])
        cw = pltpu.make_async_copy(wo_ref, wov_ref, load_sems.at[2])
        cx.start()
        cq.start()
        cw.start()

        barrier = pltpu.get_barrier_semaphore()
        for nbr in peers:
            pl.semaphore_signal(barrier, inc=1, device_id=(nbr,),
                                device_id_type=pl.DeviceIdType.MESH)

        sends = []

        def start_sends(src_ref, gather_ref, send_sems, recv_sems):
            for idx, tgt in enumerate(peers):
                for half in range(B):
                    r = pltpu.make_async_remote_copy(
                        src_ref=src_ref.at[half],
                        dst_ref=gather_ref.at[my, half],
                        send_sem=send_sems.at[idx * B + half],
                        recv_sem=recv_sems.at[my * B + half],
                        device_id=(tgt,),
                        device_id_type=pl.DeviceIdType.MESH)
                    r.start()
                    sends.append(r)

        ks_ref[...] = k_ref[...].astype(jnp.float8_e4m3fn)
        vs_ref[...] = v_ref[...].astype(jnp.bfloat16)
        pl.semaphore_wait(barrier, len(peers))
        start_sends(ks_ref, kg_ref, k_send, k_recv)
        start_sends(vs_ref, vg_ref, v_send, v_recv)

        cx.wait()
        cq.wait()
        x2 = xv_ref[...].reshape(B * S_loc, D)
        q2 = (jnp.dot(x2, wqv_ref[...],
                      preferred_element_type=jnp.float32)
              * (0.125 * 1.4426950408889634)).astype(jnp.bfloat16)
        q_all = [q2[b * S_loc:(b + 1) * S_loc] for b in range(B)]

        row = lax.broadcasted_iota(jnp.int32, (S_loc, S_loc), 0)
        col = lax.broadcasted_iota(jnp.int32, (S_loc, S_loc), 1)
        qb = (my * S_loc + row) // BLK

        acc = [[jnp.zeros((S_loc, Dh), jnp.float32)
                for _ in range(H)] for _ in range(B)]
        lsum = [[jnp.zeros((S_loc, 1), jnp.float32)
                 for _ in range(H)] for _ in range(B)]

        def block_mask(origin):
            kb = (origin * S_loc + col) // BLK
            return (qb == kb) | (kb == 0) | ((qb + kb) % 3 == 0)

        def process_half(m, b, kc2, vc2):
            for hh in range(H):
                sl = slice(hh * Dh, (hh + 1) * Dh)
                s = jnp.dot(q_all[b][:, sl], kc2[:, sl].T,
                            preferred_element_type=jnp.float32)
                e = jnp.where(m, jnp.exp2(s), 0.0)
                lsum[b][hh] += jnp.sum(e, axis=-1, keepdims=True)
                acc[b][hh] += jnp.dot(e.astype(jnp.bfloat16), vc2[:, sl],
                                      preferred_element_type=jnp.float32)

        m_own = block_mask(my)
        for b in range(B):
            process_half(m_own, b, k_ref[b].astype(jnp.bfloat16),
                         v_ref[b].astype(jnp.bfloat16))

        for origin in peers:
            m_o = block_mask(origin)
            for half in range(B):
                rk = pltpu.make_async_remote_copy(
                    src_ref=kg_ref.at[origin, half],
                    dst_ref=kg_ref.at[origin, half],
                    send_sem=k_send.at[0],
                    recv_sem=k_recv.at[origin * B + half],
                    device_id=(my,), device_id_type=pl.DeviceIdType.MESH)
                rv = pltpu.make_async_remote_copy(
                    src_ref=vg_ref.at[origin, half],
                    dst_ref=vg_ref.at[origin, half],
                    send_sem=v_send.at[0],
                    recv_sem=v_recv.at[origin * B + half],
                    device_id=(my,), device_id_type=pl.DeviceIdType.MESH)
                rk.wait_recv()
                rv.wait_recv()
                process_half(m_o, half,
                             kg_ref[origin, half].astype(jnp.bfloat16),
                             vg_ref[origin, half])

        @functools.partial(pl.run_scoped,
                           exit_sem=pltpu.SemaphoreType.REGULAR)
        def _(exit_sem):
            for nbr in peers:
                pl.semaphore_signal(exit_sem, inc=1, device_id=(nbr,),
                                    device_id_type=pl.DeviceIdType.MESH)

            cw.wait()
            for b in range(B):
                ctx = jnp.concatenate(
                    [acc[b][hh] / lsum[b][hh] for hh in range(H)], axis=-1)
                out_ref[b] = jnp.dot(ctx, wov_ref[...],
                                     preferred_element_type=jnp.float32)

            for r in sends:
                r.wait_send()
            pl.semaphore_wait(exit_sem, len(peers))

    return pl.pallas_call(
        body,
        out_shape=jax.ShapeDtypeStruct((B, S_loc, Dout), jnp.float32),
        in_specs=[
            pl.BlockSpec(memory_space=pl.ANY),
            pl.BlockSpec(memory_space=pl.ANY),
            pl.BlockSpec(memory_space=pltpu.VMEM),
            pl.BlockSpec(memory_space=pltpu.VMEM),
            pl.BlockSpec(memory_space=pl.ANY),
        ],
        out_specs=pl.BlockSpec(memory_space=pltpu.VMEM),
        scratch_shapes=[
            pltpu.VMEM((B, S_loc, D), jnp.float32),
            pltpu.VMEM((D, HD), jnp.float32),
            pltpu.VMEM((HD, Dout), jnp.float32),
            pltpu.VMEM((B, S_loc, HD), jnp.float8_e4m3fn),
            pltpu.VMEM((B, S_loc, HD), jnp.bfloat16),
            pltpu.VMEM((N_DEV, B, S_loc, HD), jnp.float8_e4m3fn),
            pltpu.VMEM((N_DEV, B, S_loc, HD), jnp.bfloat16),
            pltpu.SemaphoreType.DMA((3,)),
            pltpu.SemaphoreType.DMA((3 * B,)),
            pltpu.SemaphoreType.DMA((N_DEV * B,)),
            pltpu.SemaphoreType.DMA((3 * B,)),
            pltpu.SemaphoreType.DMA((N_DEV * B,)),
        ],
        compiler_params=pltpu.CompilerParams(collective_id=0),
    )(x, Wq, K_ext.reshape(B, S_loc, HD), V_ext.reshape(B, S_loc, HD), Wo)


# device time: 19235 ns/iter; 1.0562x vs baseline; 1.0461x over previous
import functools

import jax
import jax.numpy as jnp
from jax import lax
from jax.experimental import pallas as pl
from jax.experimental.pallas import tpu as pltpu

N_DEV = 4
BLK = 64


def kernel(x, Wq, K_ext, V_ext, Wo):
    B, S_loc, D = x.shape
    _, _, H, Dh = K_ext.shape
    HD = H * Dh
    Dout = Wo.shape[1]

    def body(x_ref, wq_ref, k_ref, v_ref, wo_ref, out_ref,
             ks_ref, vs_ref, vs8_ref, kg_ref, vg_ref, vd_ref,
             k_send, k_recv, v_send, v_recv):
        my = lax.axis_index("i")
        left = (my + N_DEV - 1) % N_DEV
        right = (my + 1) % N_DEV
        diag = (my + 2) % N_DEV
        peers = (left, right, diag)

        barrier = pltpu.get_barrier_semaphore()
        for nbr in peers:
            pl.semaphore_signal(barrier, inc=1, device_id=(nbr,),
                                device_id_type=pl.DeviceIdType.MESH)

        sends = []

        def start_send(src, dst, send_sems, sidx, recv_sems, tgt, half):
            r = pltpu.make_async_remote_copy(
                src_ref=src, dst_ref=dst,
                send_sem=send_sems.at[sidx * B + half],
                recv_sem=recv_sems.at[my * B + half],
                device_id=(tgt,), device_id_type=pl.DeviceIdType.MESH)
            r.start()
            sends.append(r)

        ks_ref[...] = k_ref[...].astype(jnp.float8_e4m3fn)
        vs_ref[...] = v_ref[...].astype(jnp.bfloat16)
        vs8_ref[...] = v_ref[...].astype(jnp.float8_e4m3fn)
        pl.semaphore_wait(barrier, len(peers))
        for half in range(B):
            for idx, tgt in enumerate(peers):
                start_send(ks_ref.at[half], kg_ref.at[my, half],
                           k_send, idx, k_recv, tgt, half)
            start_send(vs_ref.at[half], vg_ref.at[my, half],
                       v_send, 0, v_recv, left, half)
            start_send(vs_ref.at[half], vg_ref.at[my, half],
                       v_send, 1, v_recv, right, half)
            start_send(vs8_ref.at[half], vd_ref.at[half],
                       v_send, 2, v_recv, diag, half)

        x2 = x_ref[...].reshape(B * S_loc, D)
        q2 = (jnp.dot(x2, wq_ref[...],
                      preferred_element_type=jnp.float32)
              * (0.125 * 1.4426950408889634)).astype(jnp.bfloat16)
        q_all = [q2[b * S_loc:(b + 1) * S_loc] for b in range(B)]

        row = lax.broadcasted_iota(jnp.int32, (S_loc, S_loc), 0)
        col = lax.broadcasted_iota(jnp.int32, (S_loc, S_loc), 1)
        qb = (my * S_loc + row) // BLK

        acc = [[jnp.zeros((S_loc, Dh), jnp.float32)
                for _ in range(H)] for _ in range(B)]
        lsum = [[jnp.zeros((S_loc, 1), jnp.float32)
                 for _ in range(H)] for _ in range(B)]

        def block_mask(origin):
            kb = (origin * S_loc + col) // BLK
            return (qb == kb) | (kb == 0) | ((qb + kb) % 3 == 0)

        def process_half(m, b, kc2, vc2):
            for hh in range(H):
                sl = slice(hh * Dh, (hh + 1) * Dh)
                s = jnp.dot(q_all[b][:, sl], kc2[:, sl].T,
                            preferred_element_type=jnp.float32)
                e = jnp.where(m, jnp.exp2(s), 0.0)
                lsum[b][hh] += jnp.sum(e, axis=-1, keepdims=True)
                acc[b][hh] += jnp.dot(e.astype(jnp.bfloat16), vc2[:, sl],
                                      preferred_element_type=jnp.float32)

        m_own = block_mask(my)
        for b in range(B):
            process_half(m_own, b, k_ref[b].astype(jnp.bfloat16),
                         v_ref[b].astype(jnp.bfloat16))

        for is_diag, origin in ((False, left), (False, right), (True, diag)):
            m_o = block_mask(origin)
            for half in range(B):
                v_dst = vd_ref.at[half] if is_diag \
                    else vg_ref.at[origin, half]
                rk = pltpu.make_async_remote_copy(
                    src_ref=kg_ref.at[origin, half],
                    dst_ref=kg_ref.at[origin, half],
                    send_sem=k_send.at[0],
                    recv_sem=k_recv.at[origin * B + half],
                    device_id=(my,), device_id_type=pl.DeviceIdType.MESH)
                rv = pltpu.make_async_remote_copy(
                    src_ref=v_dst, dst_ref=v_dst,
                    send_sem=v_send.at[0],
                    recv_sem=v_recv.at[origin * B + half],
                    device_id=(my,), device_id_type=pl.DeviceIdType.MESH)
                rk.wait_recv()
                rv.wait_recv()
                vc2 = vd_ref[half].astype(jnp.bfloat16) if is_diag \
                    else vg_ref[origin, half]
                process_half(m_o, half,
                             kg_ref[origin, half].astype(jnp.bfloat16),
                             vc2)

        @functools.partial(pl.run_scoped,
                           exit_sem=pltpu.SemaphoreType.REGULAR)
        def _(exit_sem):
            for nbr in peers:
                pl.semaphore_signal(exit_sem, inc=1, device_id=(nbr,),
                                    device_id_type=pl.DeviceIdType.MESH)

            for b in range(B):
                ctx = jnp.concatenate(
                    [acc[b][hh] / lsum[b][hh] for hh in range(H)], axis=-1)
                out_ref[b] = jnp.dot(ctx, wo_ref[...],
                                     preferred_element_type=jnp.float32)

            for r in sends:
                r.wait_send()
            pl.semaphore_wait(exit_sem, len(peers))

    return pl.pallas_call(
        body,
        out_shape=jax.ShapeDtypeStruct((B, S_loc, Dout), jnp.float32),
        in_specs=[pl.BlockSpec(memory_space=pltpu.VMEM)] * 5,
        out_specs=pl.BlockSpec(memory_space=pltpu.VMEM),
        scratch_shapes=[
            pltpu.VMEM((B, S_loc, HD), jnp.float8_e4m3fn),
            pltpu.VMEM((B, S_loc, HD), jnp.bfloat16),
            pltpu.VMEM((B, S_loc, HD), jnp.float8_e4m3fn),
            pltpu.VMEM((N_DEV, B, S_loc, HD), jnp.float8_e4m3fn),
            pltpu.VMEM((N_DEV, B, S_loc, HD), jnp.bfloat16),
            pltpu.VMEM((B, S_loc, HD), jnp.float8_e4m3fn),
            pltpu.SemaphoreType.DMA((3 * B,)),
            pltpu.SemaphoreType.DMA((N_DEV * B,)),
            pltpu.SemaphoreType.DMA((3 * B,)),
            pltpu.SemaphoreType.DMA((N_DEV * B,)),
        ],
        compiler_params=pltpu.CompilerParams(collective_id=0),
    )(x, Wq, K_ext.reshape(B, S_loc, HD), V_ext.reshape(B, S_loc, HD), Wo)
